# Initial kernel scaffold; baseline (speedup 1.0000x reference)
import jax
import jax.numpy as jnp
from jax.experimental import pallas as pl


def kernel(x, edge_index, edge_weight, W0, b0, W1, b1, W2, b2, W3, b3):
    raise NotImplementedError("write your pallas kernel here")


# speedup vs baseline: 4.9107x; 4.9107x over previous
import functools

import jax
import jax.numpy as jnp
from jax import lax
from jax.experimental import pallas as pl
from jax.experimental.pallas import tpu as pltpu
from jax.experimental.pallas import tpu_sc as plsc

N = 10000
E = 160000
D = 256
HALF = 128
NC, NS, L = 2, 16, 16

NPAD = 10240
SL = NPAD // NS
EEXT = 172032
PAD = EEXT - E

DEG_CHUNK = 128
NRM_CHUNK = 1000
AGG_CHUNK = 128
EPT = EEXT // NS
NCH = EPT // AGG_CHUNK
OPT = N // NS

_mesh = plsc.VectorSubcoreMesh(core_axis_name="c", subcore_axis_name="s")
_sc_params = pltpu.CompilerParams(needs_layout_passes=False)

_Z16F = lambda: jnp.zeros((L,), jnp.float32)


def _prologue_body(row_hbm, col_hbm, ew_hbm, norm_hbm,
                   colbuf, ewbuf, evbuf, deg_sp, dinv_sh,
                   tmp, dinvsl, d2sl, dinvfull, rbuf, cbuf, ebuf, nbuf, zpad):
    cid = lax.axis_index("c")
    sid = lax.axis_index("s")
    z16 = _Z16F()

    def _zz(i, _):
        tmp[i, :] = z16
        return 0
    lax.fori_loop(0, SL, _zz, 0)
    pltpu.sync_copy(tmp, deg_sp.at[pl.ds(sid * SL, SL)])
    plsc.subcore_barrier()

    ebase = sid * EPT

    def _deg_chunk(c, _):
        b = ebase + c * DEG_CHUNK
        pltpu.sync_copy(col_hbm.at[pl.ds(b, DEG_CHUNK)], colbuf.at[0])
        pltpu.sync_copy(ew_hbm.at[pl.ds(b, DEG_CHUNK)], ewbuf)

        def _grp(g, _):
            ew16 = ewbuf[pl.ds(g * L, L)]
            for lane in range(L):
                evbuf[g * L + lane, :] = jnp.broadcast_to(ew16[lane], (L,))
            return 0
        lax.fori_loop(0, DEG_CHUNK // L, _grp, 0)
        pltpu.sync_copy(evbuf, deg_sp.at[colbuf.at[0]], add=True)
        return 0
    lax.fori_loop(0, EPT // DEG_CHUNK, _deg_chunk, 0)
    plsc.subcore_barrier()

    pltpu.sync_copy(deg_sp.at[pl.ds(sid * SL, SL)], tmp)
    iot = lax.iota(jnp.int32, L)
    zidx = jnp.zeros((L,), jnp.int32)

    def _red(j, _):
        rowidx = j * L + iot
        deg = plsc.load_gather(tmp, [rowidx, zidx]) + 1.0
        bi = plsc.bitcast(deg, jnp.int32)
        y = plsc.bitcast(jnp.int32(0x5F3759DF) - lax.shift_right_logical(bi, 1),
                         jnp.float32)
        for _unused in range(3):
            y = y * (1.5 - 0.5 * deg * y * y)
        gidx = sid * SL + j * L + iot
        dinvsl[pl.ds(j * L, L)] = y
        d2sl[pl.ds(j * L, L)] = jnp.where(gidx < N, y * y, 0.0)
        return 0
    lax.fori_loop(0, SL // L, _red, 0)

    pltpu.sync_copy(dinvsl, dinv_sh.at[pl.ds(sid * SL, SL)])

    @pl.when(cid == 0)
    def _():
        pltpu.sync_copy(d2sl, norm_hbm.at[pl.ds(E + sid * SL, SL)])

    @pl.when((cid == 0) & (sid == 0))
    def _():
        def _zp(i, _):
            zpad[pl.ds(i * L, L)] = z16
            return 0
        lax.fori_loop(0, (PAD - NPAD) // L, _zp, 0)
        pltpu.sync_copy(zpad, norm_hbm.at[pl.ds(E + NPAD, PAD - NPAD)])

    plsc.subcore_barrier()
    pltpu.sync_copy(dinv_sh, dinvfull)

    wid = cid * NS + sid
    wbase = wid * (E // (NC * NS))

    def _nrm_chunk(c, _):
        b = wbase + c * NRM_CHUNK
        pltpu.sync_copy(row_hbm.at[pl.ds(b, NRM_CHUNK)], rbuf)
        pltpu.sync_copy(col_hbm.at[pl.ds(b, NRM_CHUNK)], cbuf)
        pltpu.sync_copy(ew_hbm.at[pl.ds(b, NRM_CHUNK)], ebuf)

        def _one(o):
            r = rbuf[pl.ds(o, L)]
            cc = cbuf[pl.ds(o, L)]
            w = ebuf[pl.ds(o, L)]
            dr = plsc.load_gather(dinvfull, [r])
            dc = plsc.load_gather(dinvfull, [cc])
            nbuf[pl.ds(o, L)] = dr * w * dc

        def _gi(i, _):
            _one(i * L)
            return 0
        lax.fori_loop(0, NRM_CHUNK // L, _gi, 0)
        _one(NRM_CHUNK - L)
        pltpu.sync_copy(nbuf, norm_hbm.at[pl.ds(b, NRM_CHUNK)])
        return 0
    lax.fori_loop(0, (E // (NC * NS)) // NRM_CHUNK, _nrm_chunk, 0)


_prologue = pl.kernel(
    _prologue_body,
    out_type=jax.ShapeDtypeStruct((EEXT,), jnp.float32),
    mesh=_mesh,
    compiler_params=_sc_params,
    scratch_types=[
        pltpu.VMEM((1, DEG_CHUNK), jnp.int32),
        pltpu.VMEM((DEG_CHUNK,), jnp.float32),
        pltpu.VMEM((DEG_CHUNK, L), jnp.float32),
        pltpu.VMEM_SHARED((NPAD, L), jnp.float32),
        pltpu.VMEM_SHARED((NPAD,), jnp.float32),
        pltpu.VMEM((SL, L), jnp.float32),
        pltpu.VMEM((SL,), jnp.float32),
        pltpu.VMEM((SL,), jnp.float32),
        pltpu.VMEM((NPAD,), jnp.float32),
        pltpu.VMEM((NRM_CHUNK,), jnp.int32),
        pltpu.VMEM((NRM_CHUNK,), jnp.int32),
        pltpu.VMEM((NRM_CHUNK,), jnp.float32),
        pltpu.VMEM((NRM_CHUNK,), jnp.float32),
        pltpu.VMEM((PAD - NPAD,), jnp.float32),
    ],
)


def _agg_body(tlo, thi, row_hbm, col_hbm, norm_hbm, out_lo, out_hi,
              rowbuf, colbuf, normbuf, rows, acc, sem):
    cid = lax.axis_index("c")
    sid = lax.axis_index("s")
    z16 = _Z16F()

    def _zr(i, _):
        e = i // (HALF // L)
        j = i % (HALF // L)
        rows[e, pl.ds(j * L, L)] = z16
        return 0
    lax.fori_loop(0, AGG_CHUNK * (HALF // L), _zr, 0)
    for k in range(SL // AGG_CHUNK):
        pltpu.sync_copy(rows, acc.at[pl.ds(sid * SL + k * AGG_CHUNK, AGG_CHUNK)])
    plsc.subcore_barrier()

    def _chunk(c, _):
        b = sid * EPT + c * AGG_CHUNK
        pltpu.sync_copy(row_hbm.at[pl.ds(b, AGG_CHUNK)], rowbuf)
        pltpu.sync_copy(col_hbm.at[pl.ds(b, AGG_CHUNK)], colbuf.at[0])
        pltpu.sync_copy(norm_hbm.at[pl.ds(b, AGG_CHUNK)], normbuf)

        @pl.when(cid == 0)
        def _():
            pltpu.async_copy(tlo.at[rowbuf], rows, sem).wait()

        @pl.when(cid == 1)
        def _():
            pltpu.async_copy(thi.at[rowbuf], rows, sem).wait()

        def _scale(g, _):
            nv16 = normbuf[pl.ds(g * L, L)]
            for lane in range(L):
                nv = jnp.broadcast_to(nv16[lane], (L,))
                e = g * L + lane
                for j in range(HALF // L):
                    rows[e, pl.ds(j * L, L)] = rows[e, pl.ds(j * L, L)] * nv
            return 0
        lax.fori_loop(0, AGG_CHUNK // L, _scale, 0)

        pltpu.sync_copy(rows, acc.at[colbuf.at[0]], add=True)
        return 0
    lax.fori_loop(0, NCH, _chunk, 0)

    plsc.subcore_barrier()
    ob = sid * SL

    @pl.when(cid == 0)
    def _():
        pltpu.sync_copy(acc.at[pl.ds(ob, SL)], out_lo.at[pl.ds(ob, SL)])

    @pl.when(cid == 1)
    def _():
        pltpu.sync_copy(acc.at[pl.ds(ob, SL)], out_hi.at[pl.ds(ob, SL)])


_agg = pl.kernel(
    _agg_body,
    out_type=(jax.ShapeDtypeStruct((NPAD, HALF), jnp.float32),
              jax.ShapeDtypeStruct((NPAD, HALF), jnp.float32)),
    mesh=_mesh,
    compiler_params=_sc_params,
    scratch_types=[
        pltpu.VMEM((AGG_CHUNK,), jnp.int32),
        pltpu.VMEM((1, AGG_CHUNK), jnp.int32),
        pltpu.VMEM((AGG_CHUNK,), jnp.float32),
        pltpu.VMEM((AGG_CHUNK, HALF), jnp.float32),
        pltpu.VMEM_SHARED((NPAD, HALF), jnp.float32),
        pltpu.SemaphoreType.DMA,
    ],
)


BN = 400
GRID = N // BN


def _mm0_body(x_ref, w_ref, lo_ref, hi_ref):
    y = jnp.dot(x_ref[...], w_ref[...], preferred_element_type=jnp.float32,
                precision=lax.Precision.HIGHEST)
    lo_ref[...] = y[:, :HALF]
    hi_ref[...] = y[:, HALF:]


def _combine_body(alo_ref, ahi_ref, b_ref, w_ref, lo_ref, hi_ref):
    t = jnp.concatenate([alo_ref[...], ahi_ref[...]], axis=1) + b_ref[...]
    t = jnp.maximum(t, 0.0)
    y = jnp.dot(t, w_ref[...], preferred_element_type=jnp.float32,
                precision=lax.Precision.HIGHEST)
    lo_ref[...] = y[:, :HALF]
    hi_ref[...] = y[:, HALF:]


def _final_body(alo_ref, ahi_ref, b_ref, out_ref):
    out_ref[...] = (jnp.concatenate([alo_ref[...], ahi_ref[...]], axis=1)
                    + b_ref[...])


_half_spec = pl.BlockSpec((BN, HALF), lambda i: (i, 0))
_full_spec = pl.BlockSpec((BN, D), lambda i: (i, 0))
_w_spec = pl.BlockSpec((D, D), lambda i: (0, 0))
_b_spec = pl.BlockSpec((1, D), lambda i: (0, 0))
_half_out = (jax.ShapeDtypeStruct((N, HALF), jnp.float32),
             jax.ShapeDtypeStruct((N, HALF), jnp.float32))

_mm0 = pl.pallas_call(
    _mm0_body, grid=(GRID,),
    in_specs=[_full_spec, _w_spec],
    out_specs=(_half_spec, _half_spec),
    out_shape=_half_out,
)

_combine = pl.pallas_call(
    _combine_body, grid=(GRID,),
    in_specs=[_half_spec, _half_spec, _b_spec, _w_spec],
    out_specs=(_half_spec, _half_spec),
    out_shape=_half_out,
)

_final = pl.pallas_call(
    _final_body, grid=(GRID,),
    in_specs=[_half_spec, _half_spec, _b_spec],
    out_specs=_full_spec,
    out_shape=jax.ShapeDtypeStruct((N, D), jnp.float32),
)


def kernel(x, edge_index, edge_weight, W0, b0, W1, b1, W2, b2, W3, b3):
    row = edge_index[0].astype(jnp.int32)
    col = edge_index[1].astype(jnp.int32)
    ew = edge_weight.astype(jnp.float32)

    ar = jnp.arange(PAD, dtype=jnp.int32)
    selfidx = jnp.where(ar < N, ar, 0)
    row_e = jnp.concatenate([row, selfidx])
    col_e = jnp.concatenate([col, selfidx])
    ew_e = jnp.concatenate([ew, jnp.zeros((PAD,), jnp.float32)])

    norm = _prologue(row_e, col_e, ew_e)

    b0r = b0.reshape(1, D)
    b1r = b1.reshape(1, D)
    b2r = b2.reshape(1, D)
    b3r = b3.reshape(1, D)

    h_lo, h_hi = _mm0(x, W0)
    a_lo, a_hi = _agg(h_lo, h_hi, row_e, col_e, norm)
    h_lo, h_hi = _combine(a_lo, a_hi, b0r, W1)
    a_lo, a_hi = _agg(h_lo, h_hi, row_e, col_e, norm)
    h_lo, h_hi = _combine(a_lo, a_hi, b1r, W2)
    a_lo, a_hi = _agg(h_lo, h_hi, row_e, col_e, norm)
    h_lo, h_hi = _combine(a_lo, a_hi, b2r, W3)
    a_lo, a_hi = _agg(h_lo, h_hi, row_e, col_e, norm)
    return _final(a_lo, a_hi, b3r)


# speedup vs baseline: 6.7544x; 1.3754x over previous
import functools

import jax
import jax.numpy as jnp
from jax import lax
from jax.experimental import pallas as pl
from jax.experimental.pallas import tpu as pltpu
from jax.experimental.pallas import tpu_sc as plsc

N = 10000
E = 160000
D = 256
HALF = 128
NC, NS, L = 2, 16, 16

NPAD = 10240
SL = NPAD // NS
EEXT = 172032
PAD = EEXT - E

DEG_CHUNK = 128
NRM_CHUNK = 1000
AGG_CHUNK = 128
EPT = EEXT // NS
NCH = EPT // AGG_CHUNK
JG = HALF // L

_mesh = plsc.VectorSubcoreMesh(core_axis_name="c", subcore_axis_name="s")
_sc_params = pltpu.CompilerParams(needs_layout_passes=False)

_Z16F = lambda: jnp.zeros((L,), jnp.float32)


def _prologue_body(row_hbm, col_hbm, ew_hbm, norm_hbm,
                   colbuf, ewbuf, evbuf, deg_sp, dinv_sh,
                   tmp, dinvsl, d2sl, dinvfull, rbuf, cbuf, ebuf, nbuf, zpad):
    cid = lax.axis_index("c")
    sid = lax.axis_index("s")
    z16 = _Z16F()

    def _zz(i, _):
        tmp[i, :] = z16
        return 0
    lax.fori_loop(0, SL, _zz, 0)
    pltpu.sync_copy(tmp, deg_sp.at[pl.ds(sid * SL, SL)])
    plsc.subcore_barrier()

    ebase = sid * EPT

    def _deg_chunk(c, _):
        b = ebase + c * DEG_CHUNK
        pltpu.sync_copy(col_hbm.at[pl.ds(b, DEG_CHUNK)], colbuf.at[0])
        pltpu.sync_copy(ew_hbm.at[pl.ds(b, DEG_CHUNK)], ewbuf)

        def _grp(g, _):
            ew16 = ewbuf[pl.ds(g * L, L)]
            for lane in range(L):
                evbuf[g * L + lane, :] = jnp.broadcast_to(ew16[lane], (L,))
            return 0
        lax.fori_loop(0, DEG_CHUNK // L, _grp, 0)
        pltpu.sync_copy(evbuf, deg_sp.at[colbuf.at[0]], add=True)
        return 0
    lax.fori_loop(0, EPT // DEG_CHUNK, _deg_chunk, 0)
    plsc.subcore_barrier()

    pltpu.sync_copy(deg_sp.at[pl.ds(sid * SL, SL)], tmp)
    iot = lax.iota(jnp.int32, L)
    zidx = jnp.zeros((L,), jnp.int32)

    def _red(j, _):
        rowidx = j * L + iot
        deg = plsc.load_gather(tmp, [rowidx, zidx]) + 1.0
        bi = plsc.bitcast(deg, jnp.int32)
        y = plsc.bitcast(jnp.int32(0x5F3759DF) - lax.shift_right_logical(bi, 1),
                         jnp.float32)
        for _unused in range(3):
            y = y * (1.5 - 0.5 * deg * y * y)
        gidx = sid * SL + j * L + iot
        dinvsl[pl.ds(j * L, L)] = y
        d2sl[pl.ds(j * L, L)] = jnp.where(gidx < N, y * y, 0.0)
        return 0
    lax.fori_loop(0, SL // L, _red, 0)

    pltpu.sync_copy(dinvsl, dinv_sh.at[pl.ds(sid * SL, SL)])

    @pl.when(cid == 0)
    def _():
        pltpu.sync_copy(d2sl, norm_hbm.at[pl.ds(E + sid * SL, SL)])

    @pl.when((cid == 0) & (sid == 0))
    def _():
        def _zp(i, _):
            zpad[pl.ds(i * L, L)] = z16
            return 0
        lax.fori_loop(0, (PAD - NPAD) // L, _zp, 0)
        pltpu.sync_copy(zpad, norm_hbm.at[pl.ds(E + NPAD, PAD - NPAD)])

    plsc.subcore_barrier()
    pltpu.sync_copy(dinv_sh, dinvfull)

    wid = cid * NS + sid
    wbase = wid * (E // (NC * NS))

    def _nrm_chunk(c, _):
        b = wbase + c * NRM_CHUNK
        pltpu.sync_copy(row_hbm.at[pl.ds(b, NRM_CHUNK)], rbuf)
        pltpu.sync_copy(col_hbm.at[pl.ds(b, NRM_CHUNK)], cbuf)
        pltpu.sync_copy(ew_hbm.at[pl.ds(b, NRM_CHUNK)], ebuf)

        def _one(o):
            r = rbuf[pl.ds(o, L)]
            cc = cbuf[pl.ds(o, L)]
            w = ebuf[pl.ds(o, L)]
            dr = plsc.load_gather(dinvfull, [r])
            dc = plsc.load_gather(dinvfull, [cc])
            nbuf[pl.ds(o, L)] = dr * w * dc

        def _gi(i, _):
            _one(i * L)
            return 0
        lax.fori_loop(0, NRM_CHUNK // L, _gi, 0)
        _one(NRM_CHUNK - L)
        pltpu.sync_copy(nbuf, norm_hbm.at[pl.ds(b, NRM_CHUNK)])
        return 0
    lax.fori_loop(0, (E // (NC * NS)) // NRM_CHUNK, _nrm_chunk, 0)


_prologue = pl.kernel(
    _prologue_body,
    out_type=jax.ShapeDtypeStruct((EEXT,), jnp.float32),
    mesh=_mesh,
    compiler_params=_sc_params,
    scratch_types=[
        pltpu.VMEM((1, DEG_CHUNK), jnp.int32),
        pltpu.VMEM((DEG_CHUNK,), jnp.float32),
        pltpu.VMEM((DEG_CHUNK, L), jnp.float32),
        pltpu.VMEM_SHARED((NPAD, L), jnp.float32),
        pltpu.VMEM_SHARED((NPAD,), jnp.float32),
        pltpu.VMEM((SL, L), jnp.float32),
        pltpu.VMEM((SL,), jnp.float32),
        pltpu.VMEM((SL,), jnp.float32),
        pltpu.VMEM((NPAD,), jnp.float32),
        pltpu.VMEM((NRM_CHUNK,), jnp.int32),
        pltpu.VMEM((NRM_CHUNK,), jnp.int32),
        pltpu.VMEM((NRM_CHUNK,), jnp.float32),
        pltpu.VMEM((NRM_CHUNK,), jnp.float32),
        pltpu.VMEM((PAD - NPAD,), jnp.float32),
    ],
)


def _agg_body(tlo, thi, row_hbm, col_hbm, norm_hbm, out_lo, out_hi,
              rowb, colb, nb0, nb1, rows0, rows1, acc, semI, semG):
    cid = lax.axis_index("c")
    sid = lax.axis_index("s")
    z16 = _Z16F()
    ebase = sid * EPT
    rowss = (rows0, rows1)
    nbufs = (nb0, nb1)

    def _ze(e, _):
        for j in range(JG):
            rows0[e, pl.ds(j * L, L)] = z16
        return 0
    lax.fori_loop(0, AGG_CHUNK, _ze, 0)
    for k in range(SL // AGG_CHUNK):
        pltpu.sync_copy(rows0, acc.at[pl.ds(sid * SL + k * AGG_CHUNK, AGG_CHUNK)])
    plsc.subcore_barrier()

    def fetch_idx(c, b):
        base = ebase + c * AGG_CHUNK
        pltpu.async_copy(row_hbm.at[pl.ds(base, AGG_CHUNK)], rowb.at[b], semI)
        pltpu.async_copy(col_hbm.at[pl.ds(base, AGG_CHUNK)], colb.at[b], semI)
        pltpu.async_copy(norm_hbm.at[pl.ds(base, AGG_CHUNK)], nbufs[b], semI)

    def drain_idx(b):
        pltpu.make_async_copy(row_hbm.at[pl.ds(0, AGG_CHUNK)], rowb.at[b],
                              semI).wait()
        pltpu.make_async_copy(col_hbm.at[pl.ds(0, AGG_CHUNK)], colb.at[b],
                              semI).wait()
        pltpu.make_async_copy(norm_hbm.at[pl.ds(0, AGG_CHUNK)], nbufs[b],
                              semI).wait()

    def start_gather(b):
        @pl.when(cid == 0)
        def _():
            pltpu.async_copy(tlo.at[rowb.at[b]], rowss[b], semG)

        @pl.when(cid == 1)
        def _():
            pltpu.async_copy(thi.at[rowb.at[b]], rowss[b], semG)

    def drain_gather(b):
        pltpu.make_async_copy(tlo.at[rowb.at[b]], rowss[b], semG).wait()

    def process(b):
        rows, nb = rowss[b], nbufs[b]

        def _e(e, _):
            nv = plsc.load_gather(nb, [jnp.broadcast_to(e, (L,))])
            for j in range(JG):
                rows[e, pl.ds(j * L, L)] = rows[e, pl.ds(j * L, L)] * nv
            return 0
        lax.fori_loop(0, AGG_CHUNK, _e, 0)
        pltpu.sync_copy(rows, acc.at[colb.at[b]], add=True)

    fetch_idx(0, 0)
    drain_idx(0)
    start_gather(0)
    fetch_idx(1, 1)

    def _pipe(i, _):
        c0 = i * 2
        drain_gather(0)
        drain_idx(1)
        start_gather(1)
        process(0)
        fetch_idx(c0 + 2, 0)
        drain_gather(1)
        drain_idx(0)
        start_gather(0)
        process(1)
        fetch_idx(c0 + 3, 1)
        return 0
    lax.fori_loop(0, NCH // 2 - 1, _pipe, 0)

    drain_gather(0)
    drain_idx(1)
    start_gather(1)
    process(0)
    drain_gather(1)
    process(1)

    plsc.subcore_barrier()
    ob = sid * SL

    @pl.when(cid == 0)
    def _():
        pltpu.sync_copy(acc.at[pl.ds(ob, SL)], out_lo.at[pl.ds(ob, SL)])

    @pl.when(cid == 1)
    def _():
        pltpu.sync_copy(acc.at[pl.ds(ob, SL)], out_hi.at[pl.ds(ob, SL)])


_agg = pl.kernel(
    _agg_body,
    out_type=(jax.ShapeDtypeStruct((NPAD, HALF), jnp.float32),
              jax.ShapeDtypeStruct((NPAD, HALF), jnp.float32)),
    mesh=_mesh,
    compiler_params=_sc_params,
    scratch_types=[
        pltpu.VMEM((2, AGG_CHUNK), jnp.int32),
        pltpu.VMEM((2, AGG_CHUNK), jnp.int32),
        pltpu.VMEM((AGG_CHUNK,), jnp.float32),
        pltpu.VMEM((AGG_CHUNK,), jnp.float32),
        pltpu.VMEM((AGG_CHUNK, HALF), jnp.float32),
        pltpu.VMEM((AGG_CHUNK, HALF), jnp.float32),
        pltpu.VMEM_SHARED((NPAD, HALF), jnp.float32),
        pltpu.SemaphoreType.DMA,
        pltpu.SemaphoreType.DMA,
    ],
)


BN = 400
GRID = N // BN


def _mm0_body(x_ref, w_ref, lo_ref, hi_ref):
    y = jnp.dot(x_ref[...], w_ref[...], preferred_element_type=jnp.float32,
                precision=lax.Precision.HIGHEST)
    lo_ref[...] = y[:, :HALF]
    hi_ref[...] = y[:, HALF:]


def _combine_body(alo_ref, ahi_ref, b_ref, w_ref, lo_ref, hi_ref):
    t = jnp.concatenate([alo_ref[...], ahi_ref[...]], axis=1) + b_ref[...]
    t = jnp.maximum(t, 0.0)
    y = jnp.dot(t, w_ref[...], preferred_element_type=jnp.float32,
                precision=lax.Precision.HIGHEST)
    lo_ref[...] = y[:, :HALF]
    hi_ref[...] = y[:, HALF:]


def _final_body(alo_ref, ahi_ref, b_ref, out_ref):
    out_ref[...] = (jnp.concatenate([alo_ref[...], ahi_ref[...]], axis=1)
                    + b_ref[...])


_half_spec = pl.BlockSpec((BN, HALF), lambda i: (i, 0))
_full_spec = pl.BlockSpec((BN, D), lambda i: (i, 0))
_w_spec = pl.BlockSpec((D, D), lambda i: (0, 0))
_b_spec = pl.BlockSpec((1, D), lambda i: (0, 0))
_half_out = (jax.ShapeDtypeStruct((N, HALF), jnp.float32),
             jax.ShapeDtypeStruct((N, HALF), jnp.float32))

_mm0 = pl.pallas_call(
    _mm0_body, grid=(GRID,),
    in_specs=[_full_spec, _w_spec],
    out_specs=(_half_spec, _half_spec),
    out_shape=_half_out,
)

_combine = pl.pallas_call(
    _combine_body, grid=(GRID,),
    in_specs=[_half_spec, _half_spec, _b_spec, _w_spec],
    out_specs=(_half_spec, _half_spec),
    out_shape=_half_out,
)

_final = pl.pallas_call(
    _final_body, grid=(GRID,),
    in_specs=[_half_spec, _half_spec, _b_spec],
    out_specs=_full_spec,
    out_shape=jax.ShapeDtypeStruct((N, D), jnp.float32),
)


def kernel(x, edge_index, edge_weight, W0, b0, W1, b1, W2, b2, W3, b3):
    row = edge_index[0].astype(jnp.int32)
    col = edge_index[1].astype(jnp.int32)
    ew = edge_weight.astype(jnp.float32)

    ar = jnp.arange(PAD, dtype=jnp.int32)
    selfidx = jnp.where(ar < N, ar, 0)
    row_e = jnp.concatenate([row, selfidx])
    col_e = jnp.concatenate([col, selfidx])
    ew_e = jnp.concatenate([ew, jnp.zeros((PAD,), jnp.float32)])

    norm = _prologue(row_e, col_e, ew_e)

    b0r = b0.reshape(1, D)
    b1r = b1.reshape(1, D)
    b2r = b2.reshape(1, D)
    b3r = b3.reshape(1, D)

    h_lo, h_hi = _mm0(x, W0)
    a_lo, a_hi = _agg(h_lo, h_hi, row_e, col_e, norm)
    h_lo, h_hi = _combine(a_lo, a_hi, b0r, W1)
    a_lo, a_hi = _agg(h_lo, h_hi, row_e, col_e, norm)
    h_lo, h_hi = _combine(a_lo, a_hi, b1r, W2)
    a_lo, a_hi = _agg(h_lo, h_hi, row_e, col_e, norm)
    h_lo, h_hi = _combine(a_lo, a_hi, b2r, W3)
    a_lo, a_hi = _agg(h_lo, h_hi, row_e, col_e, norm)
    return _final(a_lo, a_hi, b3r)


# speedup vs baseline: 7.0950x; 1.0504x over previous
import functools

import jax
import jax.numpy as jnp
from jax import lax
from jax.experimental import pallas as pl
from jax.experimental.pallas import tpu as pltpu
from jax.experimental.pallas import tpu_sc as plsc

N = 10000
E = 160000
D = 256
HALF = 128
NC, NS, L = 2, 16, 16

NPAD = 10240
SL = NPAD // NS
EEXT = 172032
PAD = EEXT - E

DEG_CHUNK = 128
NRM_CHUNK = 1000
AGG_CHUNK = 128
EPT = EEXT // NS
NCH = EPT // AGG_CHUNK
JG = HALF // L

_mesh = plsc.VectorSubcoreMesh(core_axis_name="c", subcore_axis_name="s")
_sc_params = pltpu.CompilerParams(needs_layout_passes=False)

_Z16F = lambda: jnp.zeros((L,), jnp.float32)


def _prologue_body(row_hbm, col_hbm, ew_hbm, norm_hbm,
                   colbuf, ewbuf, evbuf, deg_sp, dinv_sh,
                   tmp, dinvsl, d2sl, dinvfull, rbuf, cbuf, ebuf, nbuf, zpad):
    cid = lax.axis_index("c")
    sid = lax.axis_index("s")
    z16 = _Z16F()

    def _zz(i, _):
        tmp[i, :] = z16
        return 0
    lax.fori_loop(0, SL, _zz, 0)
    pltpu.sync_copy(tmp, deg_sp.at[pl.ds(sid * SL, SL)])
    plsc.subcore_barrier()

    ebase = sid * EPT

    def _deg_chunk(c, _):
        b = ebase + c * DEG_CHUNK
        pltpu.sync_copy(col_hbm.at[pl.ds(b, DEG_CHUNK)], colbuf.at[0])
        pltpu.sync_copy(ew_hbm.at[pl.ds(b, DEG_CHUNK)], ewbuf)

        def _grp(g, _):
            ew16 = ewbuf[pl.ds(g * L, L)]
            for lane in range(L):
                evbuf[g * L + lane, :] = jnp.broadcast_to(ew16[lane], (L,))
            return 0
        lax.fori_loop(0, DEG_CHUNK // L, _grp, 0)
        pltpu.sync_copy(evbuf, deg_sp.at[colbuf.at[0]], add=True)
        return 0
    lax.fori_loop(0, EPT // DEG_CHUNK, _deg_chunk, 0)
    plsc.subcore_barrier()

    pltpu.sync_copy(deg_sp.at[pl.ds(sid * SL, SL)], tmp)
    iot = lax.iota(jnp.int32, L)
    zidx = jnp.zeros((L,), jnp.int32)

    def _red(j, _):
        rowidx = j * L + iot
        deg = plsc.load_gather(tmp, [rowidx, zidx]) + 1.0
        bi = plsc.bitcast(deg, jnp.int32)
        y = plsc.bitcast(jnp.int32(0x5F3759DF) - lax.shift_right_logical(bi, 1),
                         jnp.float32)
        for _unused in range(3):
            y = y * (1.5 - 0.5 * deg * y * y)
        gidx = sid * SL + j * L + iot
        dinvsl[pl.ds(j * L, L)] = y
        d2sl[pl.ds(j * L, L)] = jnp.where(gidx < N, y * y, 0.0)
        return 0
    lax.fori_loop(0, SL // L, _red, 0)

    pltpu.sync_copy(dinvsl, dinv_sh.at[pl.ds(sid * SL, SL)])

    @pl.when(cid == 0)
    def _():
        pltpu.sync_copy(d2sl, norm_hbm.at[pl.ds(E + sid * SL, SL)])

    @pl.when((cid == 0) & (sid == 0))
    def _():
        def _zp(i, _):
            zpad[pl.ds(i * L, L)] = z16
            return 0
        lax.fori_loop(0, (PAD - NPAD) // L, _zp, 0)
        pltpu.sync_copy(zpad, norm_hbm.at[pl.ds(E + NPAD, PAD - NPAD)])

    plsc.subcore_barrier()
    pltpu.sync_copy(dinv_sh, dinvfull)

    wid = cid * NS + sid
    wbase = wid * (E // (NC * NS))

    def _nrm_chunk(c, _):
        b = wbase + c * NRM_CHUNK
        pltpu.sync_copy(row_hbm.at[pl.ds(b, NRM_CHUNK)], rbuf)
        pltpu.sync_copy(col_hbm.at[pl.ds(b, NRM_CHUNK)], cbuf)
        pltpu.sync_copy(ew_hbm.at[pl.ds(b, NRM_CHUNK)], ebuf)

        def _one(o):
            r = rbuf[pl.ds(o, L)]
            cc = cbuf[pl.ds(o, L)]
            w = ebuf[pl.ds(o, L)]
            dr = plsc.load_gather(dinvfull, [r])
            dc = plsc.load_gather(dinvfull, [cc])
            nbuf[pl.ds(o, L)] = dr * w * dc

        def _gi(i, _):
            _one(i * L)
            return 0
        lax.fori_loop(0, NRM_CHUNK // L, _gi, 0)
        _one(NRM_CHUNK - L)
        pltpu.sync_copy(nbuf, norm_hbm.at[pl.ds(b, NRM_CHUNK)])
        return 0
    lax.fori_loop(0, (E // (NC * NS)) // NRM_CHUNK, _nrm_chunk, 0)


_prologue = pl.kernel(
    _prologue_body,
    out_type=jax.ShapeDtypeStruct((EEXT,), jnp.float32),
    mesh=_mesh,
    compiler_params=_sc_params,
    scratch_types=[
        pltpu.VMEM((1, DEG_CHUNK), jnp.int32),
        pltpu.VMEM((DEG_CHUNK,), jnp.float32),
        pltpu.VMEM((DEG_CHUNK, L), jnp.float32),
        pltpu.VMEM_SHARED((NPAD, L), jnp.float32),
        pltpu.VMEM_SHARED((NPAD,), jnp.float32),
        pltpu.VMEM((SL, L), jnp.float32),
        pltpu.VMEM((SL,), jnp.float32),
        pltpu.VMEM((SL,), jnp.float32),
        pltpu.VMEM((NPAD,), jnp.float32),
        pltpu.VMEM((NRM_CHUNK,), jnp.int32),
        pltpu.VMEM((NRM_CHUNK,), jnp.int32),
        pltpu.VMEM((NRM_CHUNK,), jnp.float32),
        pltpu.VMEM((NRM_CHUNK,), jnp.float32),
        pltpu.VMEM((PAD - NPAD,), jnp.float32),
    ],
)


def _agg_body(tlo, thi, row_hbm, col_hbm, norm_hbm, out_lo, out_hi,
              rowb, colb, nb0, nb1, rows0, rows1, acc, semI, semG):
    cid = lax.axis_index("c")
    sid = lax.axis_index("s")
    z16 = _Z16F()
    ebase = sid * EPT
    rowss = (rows0, rows1)
    nbufs = (nb0, nb1)

    def _ze(e, _):
        for j in range(JG):
            rows0[e, pl.ds(j * L, L)] = z16
        return 0
    lax.fori_loop(0, AGG_CHUNK, _ze, 0)
    for k in range(SL // AGG_CHUNK):
        pltpu.sync_copy(rows0, acc.at[pl.ds(sid * SL + k * AGG_CHUNK, AGG_CHUNK)])
    plsc.subcore_barrier()

    def fetch_idx(c, b):
        base = ebase + c * AGG_CHUNK
        pltpu.async_copy(row_hbm.at[pl.ds(base, AGG_CHUNK)], rowb.at[b], semI)
        pltpu.async_copy(col_hbm.at[pl.ds(base, AGG_CHUNK)], colb.at[b], semI)
        pltpu.async_copy(norm_hbm.at[pl.ds(base, AGG_CHUNK)], nbufs[b], semI)

    def drain_idx(b):
        pltpu.make_async_copy(row_hbm.at[pl.ds(0, AGG_CHUNK)], rowb.at[b],
                              semI).wait()
        pltpu.make_async_copy(col_hbm.at[pl.ds(0, AGG_CHUNK)], colb.at[b],
                              semI).wait()
        pltpu.make_async_copy(norm_hbm.at[pl.ds(0, AGG_CHUNK)], nbufs[b],
                              semI).wait()

    def start_gather(b):
        @pl.when(cid == 0)
        def _():
            pltpu.async_copy(tlo.at[rowb.at[b]], rowss[b], semG)

        @pl.when(cid == 1)
        def _():
            pltpu.async_copy(thi.at[rowb.at[b]], rowss[b], semG)

    def drain_gather(b):
        pltpu.make_async_copy(tlo.at[rowb.at[b]], rowss[b], semG).wait()

    def process(b):
        rows, nb = rowss[b], nbufs[b]

        def _e(i, _):
            for u in range(2):
                e = i * 2 + u
                nv = plsc.load_gather(nb, [jnp.broadcast_to(e, (L,))])
                for j in range(JG):
                    rows[e, pl.ds(j * L, L)] = rows[e, pl.ds(j * L, L)] * nv
            return 0
        lax.fori_loop(0, AGG_CHUNK // 2, _e, 0)
        pltpu.sync_copy(rows, acc.at[colb.at[b]], add=True)

    fetch_idx(0, 0)
    drain_idx(0)
    start_gather(0)
    fetch_idx(1, 1)

    def _pipe(i, _):
        c0 = i * 2
        drain_gather(0)
        drain_idx(1)
        start_gather(1)
        process(0)
        fetch_idx(c0 + 2, 0)
        drain_gather(1)
        drain_idx(0)
        start_gather(0)
        process(1)
        fetch_idx(c0 + 3, 1)
        return 0
    lax.fori_loop(0, NCH // 2 - 1, _pipe, 0)

    drain_gather(0)
    drain_idx(1)
    start_gather(1)
    process(0)
    drain_gather(1)
    process(1)

    plsc.subcore_barrier()
    ob = sid * SL

    @pl.when(cid == 0)
    def _():
        pltpu.sync_copy(acc.at[pl.ds(ob, SL)], out_lo.at[pl.ds(ob, SL)])

    @pl.when(cid == 1)
    def _():
        pltpu.sync_copy(acc.at[pl.ds(ob, SL)], out_hi.at[pl.ds(ob, SL)])


_agg = pl.kernel(
    _agg_body,
    out_type=(jax.ShapeDtypeStruct((NPAD, HALF), jnp.float32),
              jax.ShapeDtypeStruct((NPAD, HALF), jnp.float32)),
    mesh=_mesh,
    compiler_params=_sc_params,
    scratch_types=[
        pltpu.VMEM((2, AGG_CHUNK), jnp.int32),
        pltpu.VMEM((2, AGG_CHUNK), jnp.int32),
        pltpu.VMEM((AGG_CHUNK,), jnp.float32),
        pltpu.VMEM((AGG_CHUNK,), jnp.float32),
        pltpu.VMEM((AGG_CHUNK, HALF), jnp.float32),
        pltpu.VMEM((AGG_CHUNK, HALF), jnp.float32),
        pltpu.VMEM_SHARED((NPAD, HALF), jnp.float32),
        pltpu.SemaphoreType.DMA,
        pltpu.SemaphoreType.DMA,
    ],
)


BN = 400
GRID = N // BN


def _mm0_body(x_ref, w_ref, lo_ref, hi_ref):
    y = jnp.dot(x_ref[...], w_ref[...], preferred_element_type=jnp.float32,
                precision=lax.Precision.HIGHEST)
    lo_ref[...] = y[:, :HALF]
    hi_ref[...] = y[:, HALF:]


def _combine_body(alo_ref, ahi_ref, b_ref, w_ref, lo_ref, hi_ref):
    t = jnp.concatenate([alo_ref[...], ahi_ref[...]], axis=1) + b_ref[...]
    t = jnp.maximum(t, 0.0)
    y = jnp.dot(t, w_ref[...], preferred_element_type=jnp.float32,
                precision=lax.Precision.HIGHEST)
    lo_ref[...] = y[:, :HALF]
    hi_ref[...] = y[:, HALF:]


def _final_body(alo_ref, ahi_ref, b_ref, out_ref):
    out_ref[...] = (jnp.concatenate([alo_ref[...], ahi_ref[...]], axis=1)
                    + b_ref[...])


_half_spec = pl.BlockSpec((BN, HALF), lambda i: (i, 0))
_full_spec = pl.BlockSpec((BN, D), lambda i: (i, 0))
_w_spec = pl.BlockSpec((D, D), lambda i: (0, 0))
_b_spec = pl.BlockSpec((1, D), lambda i: (0, 0))
_half_out = (jax.ShapeDtypeStruct((N, HALF), jnp.float32),
             jax.ShapeDtypeStruct((N, HALF), jnp.float32))

_mm0 = pl.pallas_call(
    _mm0_body, grid=(GRID,),
    in_specs=[_full_spec, _w_spec],
    out_specs=(_half_spec, _half_spec),
    out_shape=_half_out,
)

_combine = pl.pallas_call(
    _combine_body, grid=(GRID,),
    in_specs=[_half_spec, _half_spec, _b_spec, _w_spec],
    out_specs=(_half_spec, _half_spec),
    out_shape=_half_out,
)

_final = pl.pallas_call(
    _final_body, grid=(GRID,),
    in_specs=[_half_spec, _half_spec, _b_spec],
    out_specs=_full_spec,
    out_shape=jax.ShapeDtypeStruct((N, D), jnp.float32),
)


def kernel(x, edge_index, edge_weight, W0, b0, W1, b1, W2, b2, W3, b3):
    row = edge_index[0].astype(jnp.int32)
    col = edge_index[1].astype(jnp.int32)
    ew = edge_weight.astype(jnp.float32)

    ar = jnp.arange(PAD, dtype=jnp.int32)
    selfidx = jnp.where(ar < N, ar, 0)
    row_e = jnp.concatenate([row, selfidx])
    col_e = jnp.concatenate([col, selfidx])
    ew_e = jnp.concatenate([ew, jnp.zeros((PAD,), jnp.float32)])

    norm = _prologue(row_e, col_e, ew_e)

    b0r = b0.reshape(1, D)
    b1r = b1.reshape(1, D)
    b2r = b2.reshape(1, D)
    b3r = b3.reshape(1, D)

    h_lo, h_hi = _mm0(x, W0)
    a_lo, a_hi = _agg(h_lo, h_hi, row_e, col_e, norm)
    h_lo, h_hi = _combine(a_lo, a_hi, b0r, W1)
    a_lo, a_hi = _agg(h_lo, h_hi, row_e, col_e, norm)
    h_lo, h_hi = _combine(a_lo, a_hi, b1r, W2)
    a_lo, a_hi = _agg(h_lo, h_hi, row_e, col_e, norm)
    h_lo, h_hi = _combine(a_lo, a_hi, b2r, W3)
    a_lo, a_hi = _agg(h_lo, h_hi, row_e, col_e, norm)
    return _final(a_lo, a_hi, b3r)
